# Initial kernel scaffold; baseline (speedup 1.0000x reference)
import jax
import jax.numpy as jnp
from jax.experimental import pallas as pl


def kernel(x, edge_index, edge_attr, u, batch, W1, b1, W2, b2):
    raise NotImplementedError("write your pallas kernel here")


# speedup vs baseline: 2.3597x; 2.3597x over previous
import jax
import jax.numpy as jnp
from jax import lax
from jax.experimental import pallas as pl
from jax.experimental.pallas import tpu as pltpu
from jax.experimental.pallas import tpu_sc as plsc

N_NODES = 10000
N_F = 256
U_F = 128
HIDDEN = 512
N_GRAPHS = 64

NC = 2
NS = 16
NW = NC * NS
LANES = 16
NG_F = N_F // LANES

S = 80
GRP = 5
NCHUNK = N_NODES // S
CHUNKS_PER_W = -(-NCHUNK // NW)
ROWS_PER_SUB = N_GRAPHS // NS

_mesh = plsc.VectorSubcoreMesh(core_axis_name="c", subcore_axis_name="s")


def _sc_segment_sums(x_hbm, batch2_hbm, out_hbm, idx_v, xbuf_v, acc_v,
                     rbuf_v, tbuf_v, stage_sh):
    cid = lax.axis_index("c")
    sid = lax.axis_index("s")
    wid = sid * NC + cid

    zero16 = jnp.zeros((LANES,), jnp.float32)
    lane = lax.iota(jnp.int32, LANES)

    def _zero_row(r, c):
        acc_v[pl.ds(r * LANES, LANES)] = zero16
        return c

    lax.fori_loop(0, N_GRAPHS * NG_F, _zero_row, 0)

    def _do_group(g, c):
        gvec = idx_v[pl.ds(g * LANES, LANES)]
        base = g * LANES
        for r in range(LANES):
            rid = gvec[jnp.full((LANES,), r, jnp.int32)]
            abase = rid * N_F + lane
            for k in range(NG_F):
                xv = xbuf_v[base + r, pl.ds(k * LANES, LANES)]
                plsc.addupdate_scatter(acc_v, [abase + (k * LANES)], xv)
        return c

    def _do_chunk(t, c):
        j = wid + t * NW

        @pl.when(j < NCHUNK)
        def _():
            pltpu.sync_copy(batch2_hbm.at[j], idx_v)
            pltpu.sync_copy(x_hbm.at[pl.ds(j * S, S)], xbuf_v)
            lax.fori_loop(0, GRP, _do_group, 0)

        return c

    lax.fori_loop(0, CHUNKS_PER_W, _do_chunk, 0)

    pltpu.sync_copy(acc_v, stage_sh.at[sid])
    plsc.subcore_barrier()

    base = sid * ROWS_PER_SUB * N_F
    for k in range(ROWS_PER_SUB * NG_F):
        rbuf_v[pl.ds(k * LANES, LANES)] = zero16
    for t in range(NS):
        pltpu.sync_copy(stage_sh.at[t, pl.ds(base, ROWS_PER_SUB * N_F)],
                        tbuf_v)
        for k in range(ROWS_PER_SUB * NG_F):
            sl = pl.ds(k * LANES, LANES)
            rbuf_v[sl] = rbuf_v[sl] + tbuf_v[sl]

    pltpu.sync_copy(rbuf_v, out_hbm.at[cid, pl.ds(base, ROWS_PER_SUB * N_F)])


_sc_call = pl.kernel(
    _sc_segment_sums,
    out_type=jax.ShapeDtypeStruct((NC, N_GRAPHS * N_F), jnp.float32),
    mesh=_mesh,
    scratch_types=[
        pltpu.VMEM((S,), jnp.int32),
        pltpu.VMEM((S, N_F), jnp.float32),
        pltpu.VMEM((N_GRAPHS * N_F,), jnp.float32),
        pltpu.VMEM((ROWS_PER_SUB * N_F,), jnp.float32),
        pltpu.VMEM((ROWS_PER_SUB * N_F,), jnp.float32),
        pltpu.VMEM_SHARED((NS, N_GRAPHS * N_F), jnp.float32),
    ],
    compiler_params=pltpu.CompilerParams(needs_layout_passes=False),
)

_BPAD_ROWS = 80


def _tc_finish(part_ref, bpad_ref, u_ref, w1_ref, b1_ref, w2_ref, b2_ref,
               out_ref):
    sums = part_ref[0] + part_ref[1]
    b = bpad_ref[:]
    gids = lax.broadcasted_iota(jnp.int32, (N_GRAPHS, 1, 1), 0)
    counts = jnp.sum((b[None, :, :] == gids).astype(jnp.float32), axis=(1, 2))
    mean = sums / jnp.maximum(counts, 1.0)[:, None]
    cat = jnp.concatenate([u_ref[:], mean], axis=1)
    h = jnp.maximum(
        jnp.dot(cat, w1_ref[:], preferred_element_type=jnp.float32)
        + b1_ref[:], 0.0)
    out_ref[:] = (jnp.dot(h, w2_ref[:], preferred_element_type=jnp.float32)
                  + b2_ref[:])


_tc_call = pl.pallas_call(
    _tc_finish,
    out_shape=jax.ShapeDtypeStruct((N_GRAPHS, U_F), jnp.float32),
)


@jax.jit
def kernel(x, edge_index, edge_attr, u, batch, W1, b1, W2, b2):
    del edge_index, edge_attr
    batch_i = batch.astype(jnp.int32)
    batch2 = batch_i.reshape(NCHUNK, S)
    npad = _BPAD_ROWS * 128 - N_NODES
    bpad = jnp.concatenate(
        [batch_i, jnp.full((npad,), N_GRAPHS, jnp.int32)]).reshape(
            _BPAD_ROWS, 128)
    partials = _sc_call(x, batch2).reshape(NC, N_GRAPHS, N_F)
    return _tc_call(partials, bpad, u, W1, b1.reshape(1, HIDDEN), W2,
                    b2.reshape(1, U_F))


# speedup vs baseline: 2.7223x; 1.1537x over previous
import jax
import jax.numpy as jnp
from jax import lax
from jax.experimental import pallas as pl
from jax.experimental.pallas import tpu as pltpu
from jax.experimental.pallas import tpu_sc as plsc

N_NODES = 10000
N_F = 256
U_F = 128
HIDDEN = 512
N_GRAPHS = 64

NC = 2
NS = 16
NW = NC * NS
LANES = 16
NG_F = N_F // LANES

S = 80
GRP = 5
NCHUNK = N_NODES // S
CHUNKS_PER_W = -(-NCHUNK // NW)
ROWS_PER_SUB = N_GRAPHS // NS

_mesh = plsc.VectorSubcoreMesh(core_axis_name="c", subcore_axis_name="s")


def _sc_segment_sums(x_hbm, batch2_hbm, out_hbm, idx_v, xbufs, acc_v,
                     rbuf_v, tbufs, stage_sh, xsems, isems, tsems):
    cid = lax.axis_index("c")
    sid = lax.axis_index("s")
    wid = sid * NC + cid

    zero16 = jnp.zeros((LANES,), jnp.float32)
    lane = lax.iota(jnp.int32, LANES)

    def _start(t, buf):
        j = wid + t * NW

        @pl.when(j < NCHUNK)
        def _():
            pltpu.async_copy(batch2_hbm.at[j], idx_v.at[buf], isems.at[buf])
            pltpu.async_copy(x_hbm.at[pl.ds(j * S, S)], xbufs.at[buf],
                             xsems.at[buf])

    _start(0, 0)

    def _zero_blk(b, c):
        for k in range(64):
            acc_v[pl.ds((b * 64 + k) * LANES, LANES)] = zero16
        return c

    lax.fori_loop(0, N_GRAPHS * NG_F // 64, _zero_blk, 0)

    def _do_group(g, buf):
        gvec = idx_v[buf, pl.ds(g * LANES, LANES)]
        base = g * LANES
        for r in range(LANES):
            rid = gvec[jnp.full((LANES,), r, jnp.int32)]
            abase = rid * N_F + lane
            for k in range(NG_F):
                xv = xbufs[buf, base + r, pl.ds(k * LANES, LANES)]
                plsc.addupdate_scatter(acc_v, [abase + (k * LANES)], xv)
        return buf

    for t in range(CHUNKS_PER_W):
        j = wid + t * NW
        buf = t % 2
        if t + 1 < CHUNKS_PER_W:
            _start(t + 1, 1 - buf)

        @pl.when(j < NCHUNK)
        def _():
            pltpu.make_async_copy(batch2_hbm.at[j], idx_v.at[buf],
                                  isems.at[buf]).wait()
            pltpu.make_async_copy(x_hbm.at[pl.ds(j * S, S)], xbufs.at[buf],
                                  xsems.at[buf]).wait()
            lax.fori_loop(0, GRP, _do_group, buf)

    pltpu.sync_copy(acc_v, stage_sh.at[sid])
    plsc.subcore_barrier()

    for p in range(2):
        row0 = sid * ROWS_PER_SUB + p * 2
        off = row0 * N_F

        def _tstart(t, buf):
            pltpu.async_copy(stage_sh.at[t, pl.ds(off, 2 * N_F)],
                             tbufs.at[buf], tsems.at[buf])

        _tstart(0, 0)
        regs = [zero16] * (2 * NG_F)
        for t in range(NS):
            buf = t % 2
            if t + 1 < NS:
                _tstart(t + 1, 1 - buf)
            pltpu.make_async_copy(stage_sh.at[t, pl.ds(off, 2 * N_F)],
                                  tbufs.at[buf], tsems.at[buf]).wait()
            for k in range(2 * NG_F):
                regs[k] = regs[k] + tbufs[buf, pl.ds(k * LANES, LANES)]
        for k in range(2 * NG_F):
            rbuf_v[pl.ds(k * LANES, LANES)] = regs[k]
        pltpu.sync_copy(rbuf_v, out_hbm.at[cid, pl.ds(off, 2 * N_F)])


_sc_call = pl.kernel(
    _sc_segment_sums,
    out_type=jax.ShapeDtypeStruct((NC, N_GRAPHS * N_F), jnp.float32),
    mesh=_mesh,
    scratch_types=[
        pltpu.VMEM((2, S), jnp.int32),
        pltpu.VMEM((2, S, N_F), jnp.float32),
        pltpu.VMEM((N_GRAPHS * N_F,), jnp.float32),
        pltpu.VMEM((2 * N_F,), jnp.float32),
        pltpu.VMEM((2, 2 * N_F), jnp.float32),
        pltpu.VMEM_SHARED((NS, N_GRAPHS * N_F), jnp.float32),
        pltpu.SemaphoreType.DMA((2,)),
        pltpu.SemaphoreType.DMA((2,)),
        pltpu.SemaphoreType.DMA((2,)),
    ],
    compiler_params=pltpu.CompilerParams(needs_layout_passes=False),
)

_BPAD_ROWS = 80


def _tc_finish(part_ref, bpad_ref, u_ref, w1_ref, b1_ref, w2_ref, b2_ref,
               out_ref):
    sums = part_ref[0] + part_ref[1]
    b = bpad_ref[:]
    gids = lax.broadcasted_iota(jnp.int32, (N_GRAPHS, 1, 1), 0)
    counts = jnp.sum((b[None, :, :] == gids).astype(jnp.float32), axis=(1, 2))
    mean = sums / jnp.maximum(counts, 1.0)[:, None]
    cat = jnp.concatenate([u_ref[:], mean], axis=1)
    h = jnp.maximum(
        jnp.dot(cat, w1_ref[:], preferred_element_type=jnp.float32)
        + b1_ref[:], 0.0)
    out_ref[:] = (jnp.dot(h, w2_ref[:], preferred_element_type=jnp.float32)
                  + b2_ref[:])


_tc_call = pl.pallas_call(
    _tc_finish,
    out_shape=jax.ShapeDtypeStruct((N_GRAPHS, U_F), jnp.float32),
)


@jax.jit
def kernel(x, edge_index, edge_attr, u, batch, W1, b1, W2, b2):
    del edge_index, edge_attr
    batch_i = batch.astype(jnp.int32)
    batch2 = batch_i.reshape(NCHUNK, S)
    npad = _BPAD_ROWS * 128 - N_NODES
    bpad = jnp.concatenate(
        [batch_i, jnp.full((npad,), N_GRAPHS, jnp.int32)]).reshape(
            _BPAD_ROWS, 128)
    partials = _sc_call(x, batch2).reshape(NC, N_GRAPHS, N_F)
    return _tc_call(partials, bpad, u, W1, b1.reshape(1, HIDDEN), W2,
                    b2.reshape(1, U_F))


# speedup vs baseline: 3.2677x; 1.2003x over previous
import jax
import jax.numpy as jnp
from jax import lax
from jax.experimental import pallas as pl
from jax.experimental.pallas import tpu as pltpu
from jax.experimental.pallas import tpu_sc as plsc

N_NODES = 10000
N_F = 256
U_F = 128
HIDDEN = 512
N_GRAPHS = 64

NC = 2
NS = 16
NW = NC * NS
LANES = 16
NG_F = N_F // LANES

S = 80
GRP = 5
NCHUNK = N_NODES // S
CHUNKS_PER_W = -(-NCHUNK // NW)

_mesh = plsc.VectorSubcoreMesh(core_axis_name="c", subcore_axis_name="s")


def _sc_segment_sums(x_hbm, batch2_hbm, out_hbm, idx_v, xbufs, acc_v,
                     xsems, isems):
    cid = lax.axis_index("c")
    sid = lax.axis_index("s")
    wid = sid * NC + cid

    zero16 = jnp.zeros((LANES,), jnp.float32)
    lane = lax.iota(jnp.int32, LANES)
    zsel = jnp.zeros((LANES,), jnp.int32)

    def _start(t, buf):
        j = wid + t * NW

        @pl.when(j < NCHUNK)
        def _():
            pltpu.async_copy(batch2_hbm.at[j], idx_v.at[buf], isems.at[buf])
            pltpu.async_copy(x_hbm.at[pl.ds(j * S, S)], xbufs.at[buf],
                             xsems.at[buf])

    _start(0, 0)

    def _zero_blk(b, c):
        for k in range(64):
            acc_v[pl.ds((b * 64 + k) * LANES, LANES)] = zero16
        return c

    lax.fori_loop(0, N_GRAPHS * NG_F // 64, _zero_blk, 0)

    def _do_group(g, buf):
        gvec = idx_v[buf, pl.ds(g * LANES, LANES)]
        base = g * LANES
        first = gvec[zsel]
        nboundary = jnp.sum((gvec != first).astype(jnp.int32))

        @pl.when(nboundary == 0)
        def _fast():
            abase = first * N_F + lane
            for k in range(NG_F):
                acc = xbufs[buf, base, pl.ds(k * LANES, LANES)]
                for r in range(1, LANES):
                    acc = acc + xbufs[buf, base + r, pl.ds(k * LANES, LANES)]
                plsc.addupdate_scatter(acc_v, [abase + (k * LANES)], acc)

        @pl.when(nboundary != 0)
        def _slow():
            for r in range(LANES):
                rid = gvec[jnp.full((LANES,), r, jnp.int32)]
                abase = rid * N_F + lane
                for k in range(NG_F):
                    xv = xbufs[buf, base + r, pl.ds(k * LANES, LANES)]
                    plsc.addupdate_scatter(acc_v, [abase + (k * LANES)], xv)

        return buf

    for t in range(CHUNKS_PER_W):
        j = wid + t * NW
        buf = t % 2
        if t + 1 < CHUNKS_PER_W:
            _start(t + 1, 1 - buf)

        @pl.when(j < NCHUNK)
        def _():
            pltpu.make_async_copy(batch2_hbm.at[j], idx_v.at[buf],
                                  isems.at[buf]).wait()
            pltpu.make_async_copy(x_hbm.at[pl.ds(j * S, S)], xbufs.at[buf],
                                  xsems.at[buf]).wait()
            lax.fori_loop(0, GRP, _do_group, buf)

    pltpu.sync_copy(acc_v, out_hbm.at[wid])


_sc_call = pl.kernel(
    _sc_segment_sums,
    out_type=jax.ShapeDtypeStruct((NW, N_GRAPHS * N_F), jnp.float32),
    mesh=_mesh,
    scratch_types=[
        pltpu.VMEM((2, S), jnp.int32),
        pltpu.VMEM((2, S, N_F), jnp.float32),
        pltpu.VMEM((N_GRAPHS * N_F,), jnp.float32),
        pltpu.SemaphoreType.DMA((2,)),
        pltpu.SemaphoreType.DMA((2,)),
    ],
    compiler_params=pltpu.CompilerParams(needs_layout_passes=False),
)

_BPAD_ROWS = 80


def _tc_finish(part_ref, bpad_ref, u_ref, w1_ref, b1_ref, w2_ref, b2_ref,
               out_ref):
    sums = jnp.sum(part_ref[:], axis=0)
    b = bpad_ref[:]
    gids = lax.broadcasted_iota(jnp.int32, (N_GRAPHS, 1, 1), 0)
    counts = jnp.sum((b[None, :, :] == gids).astype(jnp.float32), axis=(1, 2))
    mean = sums / jnp.maximum(counts, 1.0)[:, None]
    cat = jnp.concatenate([u_ref[:], mean], axis=1)
    h = jnp.maximum(
        jnp.dot(cat, w1_ref[:], preferred_element_type=jnp.float32)
        + b1_ref[:], 0.0)
    out_ref[:] = (jnp.dot(h, w2_ref[:], preferred_element_type=jnp.float32)
                  + b2_ref[:])


_tc_call = pl.pallas_call(
    _tc_finish,
    out_shape=jax.ShapeDtypeStruct((N_GRAPHS, U_F), jnp.float32),
)


@jax.jit
def kernel(x, edge_index, edge_attr, u, batch, W1, b1, W2, b2):
    del edge_index, edge_attr
    batch_i = batch.astype(jnp.int32)
    batch2 = batch_i.reshape(NCHUNK, S)
    npad = _BPAD_ROWS * 128 - N_NODES
    bpad = jnp.concatenate(
        [batch_i, jnp.full((npad,), N_GRAPHS, jnp.int32)]).reshape(
            _BPAD_ROWS, 128)
    partials = _sc_call(x, batch2).reshape(NW, N_GRAPHS, N_F)
    return _tc_call(partials, bpad, u, W1, b1.reshape(1, HIDDEN), W2,
                    b2.reshape(1, U_F))


# speedup vs baseline: 3.9576x; 1.2111x over previous
import jax
import jax.numpy as jnp
from jax import lax
from jax.experimental import pallas as pl
from jax.experimental.pallas import tpu as pltpu
from jax.experimental.pallas import tpu_sc as plsc

N_NODES = 10000
N_F = 256
U_F = 128
HIDDEN = 512
N_GRAPHS = 64

NC = 2
NS = 16
NW = NC * NS
LANES = 16
NG_F = N_F // LANES

S = 80
GRP = 5
CHUNKS_PER_W = 2
NCHUNK = NW * CHUNKS_PER_W
N_SC = NCHUNK * S
N_TC = N_NODES - N_SC

_mesh = plsc.VectorSubcoreMesh(core_axis_name="c", subcore_axis_name="s")


def _sc_segment_sums(x_hbm, batch2_hbm, out_hbm, idx_v, xbufs, acc_v,
                     xsems, isems):
    cid = lax.axis_index("c")
    sid = lax.axis_index("s")
    wid = sid * NC + cid

    zero16 = jnp.zeros((LANES,), jnp.float32)
    lane = lax.iota(jnp.int32, LANES)
    zsel = jnp.zeros((LANES,), jnp.int32)
    cols = [k * LANES + lane for k in range(NG_F)]

    def _start(t, buf):
        j = wid + t * NW
        pltpu.async_copy(batch2_hbm.at[j], idx_v.at[buf], isems.at[buf])
        pltpu.async_copy(x_hbm.at[pl.ds(j * S, S)], xbufs.at[buf],
                         xsems.at[buf])

    _start(0, 0)

    def _zero_row(b, c):
        for k in range(NG_F):
            acc_v[b, pl.ds(k * LANES, LANES)] = zero16
        return c

    lax.fori_loop(0, N_GRAPHS, _zero_row, 0)

    def _do_group(g, buf):
        gvec = idx_v[buf, pl.ds(g * LANES, LANES)]
        base = g * LANES
        first = gvec[zsel]
        nboundary = jnp.sum((gvec != first).astype(jnp.int32))

        @pl.when(nboundary == 0)
        def _fast():
            for k in range(NG_F):
                acc = xbufs[buf, base, pl.ds(k * LANES, LANES)]
                for r in range(1, LANES):
                    acc = acc + xbufs[buf, base + r, pl.ds(k * LANES, LANES)]
                plsc.addupdate_scatter(acc_v, [first, cols[k]], acc)

        @pl.when(nboundary != 0)
        def _slow():
            for r in range(LANES):
                rid = gvec[jnp.full((LANES,), r, jnp.int32)]
                for k in range(NG_F):
                    xv = xbufs[buf, base + r, pl.ds(k * LANES, LANES)]
                    plsc.addupdate_scatter(acc_v, [rid, cols[k]], xv)

        return buf

    for t in range(CHUNKS_PER_W):
        buf = t % 2
        if t + 1 < CHUNKS_PER_W:
            _start(t + 1, 1 - buf)
        pltpu.make_async_copy(batch2_hbm.at[wid + t * NW], idx_v.at[buf],
                              isems.at[buf]).wait()
        pltpu.make_async_copy(x_hbm.at[pl.ds((wid + t * NW) * S, S)],
                              xbufs.at[buf], xsems.at[buf]).wait()
        lax.fori_loop(0, GRP, _do_group, buf)

    pltpu.sync_copy(acc_v, out_hbm.at[wid])


_sc_call = pl.kernel(
    _sc_segment_sums,
    out_type=jax.ShapeDtypeStruct((NW, N_GRAPHS, N_F), jnp.float32),
    mesh=_mesh,
    scratch_types=[
        pltpu.VMEM((2, S), jnp.int32),
        pltpu.VMEM((2, S, N_F), jnp.float32),
        pltpu.VMEM((N_GRAPHS, N_F), jnp.float32),
        pltpu.SemaphoreType.DMA((2,)),
        pltpu.SemaphoreType.DMA((2,)),
    ],
    compiler_params=pltpu.CompilerParams(needs_layout_passes=False),
)

_BPAD_ROWS = 80


def _tc_tail(xt_ref, bt_ref, bpad_ref, tail_ref, inv_ref):
    bt = bt_ref[:]
    gid_row = lax.broadcasted_iota(jnp.int32, (1, N_GRAPHS), 1)
    onehot = (bt == gid_row).astype(jnp.float32)
    tail_ref[:] = lax.dot_general(
        onehot, xt_ref[:], (((0,), (0,)), ((), ())),
        preferred_element_type=jnp.float32)
    b = bpad_ref[:]
    gids = lax.broadcasted_iota(jnp.int32, (N_GRAPHS, 1, 1), 0)
    counts = jnp.sum((b[None, :, :] == gids).astype(jnp.float32), axis=(1, 2))
    inv_ref[:] = (1.0 / jnp.maximum(counts, 1.0))[:, None] + jnp.zeros(
        (N_GRAPHS, 128), jnp.float32)


_tc_tail_call = pl.pallas_call(
    _tc_tail,
    out_shape=(
        jax.ShapeDtypeStruct((N_GRAPHS, N_F), jnp.float32),
        jax.ShapeDtypeStruct((N_GRAPHS, 128), jnp.float32),
    ),
    in_specs=[
        pl.BlockSpec(memory_space=pltpu.VMEM),
        pl.BlockSpec(memory_space=pltpu.VMEM),
        pl.BlockSpec(memory_space=pltpu.VMEM),
    ],
)


def _tc_finish(part_hbm, tail_ref, inv_ref, u_ref, w1_ref, b1_ref, w2_ref,
               b2_ref, out_ref, part_v, psem):
    cp = pltpu.make_async_copy(part_hbm, part_v, psem)
    cp.start()
    cp.wait()
    sums = tail_ref[:] + jnp.sum(part_v[:], axis=0)
    mean = sums * inv_ref[:, 0:1]
    cat = jnp.concatenate([u_ref[:], mean], axis=1)
    h = jnp.maximum(
        jnp.dot(cat, w1_ref[:], preferred_element_type=jnp.float32)
        + b1_ref[:], 0.0)
    out_ref[:] = (jnp.dot(h, w2_ref[:], preferred_element_type=jnp.float32)
                  + b2_ref[:])


_tc_call = pl.pallas_call(
    _tc_finish,
    out_shape=jax.ShapeDtypeStruct((N_GRAPHS, U_F), jnp.float32),
    in_specs=[
        pl.BlockSpec(memory_space=pl.ANY),
        pl.BlockSpec(memory_space=pltpu.VMEM),
        pl.BlockSpec(memory_space=pltpu.VMEM),
        pl.BlockSpec(memory_space=pltpu.VMEM),
        pl.BlockSpec(memory_space=pltpu.VMEM),
        pl.BlockSpec(memory_space=pltpu.VMEM),
        pl.BlockSpec(memory_space=pltpu.VMEM),
        pl.BlockSpec(memory_space=pltpu.VMEM),
    ],
    scratch_shapes=[
        pltpu.VMEM((NW, N_GRAPHS, N_F), jnp.float32),
        pltpu.SemaphoreType.DMA,
    ],
)


@jax.jit
def kernel(x, edge_index, edge_attr, u, batch, W1, b1, W2, b2):
    del edge_index, edge_attr
    batch_i = batch.astype(jnp.int32)
    batch2 = batch_i[:N_SC].reshape(NCHUNK, S)
    npad = _BPAD_ROWS * 128 - N_NODES
    bpad = jnp.concatenate(
        [batch_i, jnp.full((npad,), N_GRAPHS, jnp.int32)]).reshape(
            _BPAD_ROWS, 128)
    tail, inv = _tc_tail_call(x[N_SC:], batch_i[N_SC:].reshape(N_TC, 1), bpad)
    partials = _sc_call(x[:N_SC], batch2)
    return _tc_call(partials, tail, inv, u, W1, b1.reshape(1, HIDDEN), W2,
                    b2.reshape(1, U_F))
